# baseline (speedup 1.0000x reference)
import numpy as np

import jax
import jax.numpy as jnp
from jax.experimental import pallas as pl
from jax.experimental.pallas import tpu as pltpu

TH = 12
TP = 3
HID = 32
B = 8
N = 512
BN = B * N
PK = 4
PR = BN // PK
G4 = 4 * HID * PK
HP = HID * PK

_U2 = np.zeros((TH, TH * PK, 2 * PK), np.float32)
for _t in range(TH):
    for _j in range(2):
        for _s in range(PK):
            _U2[_t, TH * _s + (2 * _t + _j) % TH, 2 * _s + _j] = 1.0


def _fused_kernel(a_ref, xf_ref, w0_ref, w1_ref, w2_ref, bg_ref, wih0t_ref,
                  whh0t_ref, b0r_ref, wih1t_ref, whh1t_ref, b1r_ref, wfct_ref,
                  u2_ref, out_ref, z1_ref):
    f32 = jnp.float32
    hp = jax.lax.Precision.HIGHEST

    def zeros(r, c):
        return jnp.zeros((r, c), f32) if c > 0 else None

    def lane_concat(pieces):
        return jnp.concatenate([p for p in pieces if p is not None], axis=1)

    def kron_eye(w):
        return jnp.concatenate([
            lane_concat([zeros(TH, TH * b), w, zeros(TH, TH * (B - 1 - b))])
            for b in range(B)], axis=0)

    bws = [kron_eye(w0_ref[...]), kron_eye(w1_ref[...]), kron_eye(w2_ref[...])]
    bg = jnp.concatenate([bg_ref[...]] * B, axis=1)

    def pack_rows(wt, nrows):
        blocks = []
        for s in range(PK):
            lanes = []
            for g in range(4):
                lanes.append(zeros(nrows, HID * s))
                lanes.append(wt[:, g * HID:(g + 1) * HID])
                lanes.append(zeros(nrows, HID * (PK - 1 - s)))
            blocks.append(lane_concat(lanes))
        return jnp.concatenate(blocks, axis=0)

    bwh0 = pack_rows(whh0t_ref[...], HID)
    wz1 = jnp.concatenate([pack_rows(wih1t_ref[...], HID),
                           pack_rows(whh1t_ref[...], HID)], axis=0)

    def pack_bias(br):
        return jnp.concatenate(
            [br[:, g * HID:(g + 1) * HID] for g in range(4) for _ in range(PK)],
            axis=1)

    b0 = pack_bias(b0r_ref[...])
    b1 = pack_bias(b1r_ref[...])

    vq = pack_rows(wih0t_ref[...], 2)
    selqs = [jnp.dot(u2_ref[t], vq, preferred_element_type=f32)
             for t in range(TH)]

    wfct = wfct_ref[...]
    bwfc = jnp.concatenate([
        lane_concat([zeros(HID, s), wfct, zeros(HID, PK - 1 - s)])
        for s in range(PK)], axis=0)

    xf = xf_ref[...]
    xn = jnp.concatenate([xf[b * N:(b + 1) * N, :] for b in range(B)], axis=1)

    a = a_ref[...]
    at = a.T
    deg = jnp.sum(a, axis=1, keepdims=True)
    dinv = jnp.where(deg > 0, jax.lax.rsqrt(deg), 0.0)
    t0 = xn
    t1 = -(dinv * jnp.dot(at, dinv * t0, preferred_element_type=f32, precision=hp))
    t2 = -2.0 * (dinv * jnp.dot(at, dinv * t1, preferred_element_type=f32, precision=hp)) - t0
    hn = (jnp.dot(t0, bws[0], preferred_element_type=f32)
          + jnp.dot(t1, bws[1], preferred_element_type=f32)
          + jnp.dot(t2, bws[2], preferred_element_type=f32)
          + bg)

    half = B * TH // 2
    vcat = jnp.concatenate([xn[:, :half], xn[:, half:]], axis=0)
    hcat = jnp.concatenate([hn[:, :half], hn[:, half:]], axis=0)

    def gates_to_hc(gates, c):
        sig3 = 0.5 * jnp.tanh(0.5 * gates[:, 0 * HP:2 * HP]) + 0.5
        i = sig3[:, 0 * HP:1 * HP]
        f = sig3[:, 1 * HP:2 * HP]
        g = jnp.tanh(gates[:, 2 * HP:3 * HP])
        o = 0.5 * jnp.tanh(0.5 * gates[:, 3 * HP:4 * HP]) + 0.5
        c = f * c + i * g
        h = o * jnp.tanh(c)
        return h, c

    h = jnp.zeros((PR, HP), f32)
    c = jnp.zeros((PR, HP), f32)
    z1_ref[:, HP:2 * HP] = h
    for t in range(TH):
        src = vcat if t < TH // 2 else hcat
        gates = (jnp.dot(src, selqs[t], preferred_element_type=f32)
                 + jnp.dot(h, bwh0, preferred_element_type=f32) + b0)
        h, c = gates_to_hc(gates, c)
        z1_ref[:, t * 2 * HP:t * 2 * HP + HP] = h

    c = jnp.zeros((PR, HP), f32)
    for t in range(TH):
        zt = z1_ref[:, t * 2 * HP:(t + 1) * 2 * HP]
        gates = jnp.dot(zt, wz1, preferred_element_type=f32) + b1
        h, c = gates_to_hc(gates, c)
        if t + 1 < TH:
            z1_ref[:, (t + 1) * 2 * HP + HP:(t + 2) * 2 * HP] = h
        if t >= TH - TP:
            k = t - (TH - TP)
            out_ref[:, k * PK:(k + 1) * PK] = jnp.dot(
                h, bwfc, preferred_element_type=f32)


def kernel(X, A, W0, W1, W2, b_gcn, Wih0, Whh0, bih0, bhh0,
           Wih1, Whh1, bih1, bhh1, Wfc, bfc):
    f32 = jnp.float32
    Xf = X.reshape(BN, TH)
    bgr = b_gcn.reshape(1, TH)
    wih0t = Wih0.T
    whh0t = Whh0.T
    b0r = (bih0 + bhh0).reshape(1, 4 * HID)
    wih1t = Wih1.T
    whh1t = Whh1.T
    b1r = (bih1 + bhh1).reshape(1, 4 * HID)
    wfct = Wfc.reshape(HID, 1)

    out = pl.pallas_call(
        _fused_kernel,
        out_shape=jax.ShapeDtypeStruct((PR, TP * PK), f32),
        scratch_shapes=[pltpu.VMEM((PR, TH * 2 * HP), f32)],
    )(A, Xf, W0, W1, W2, bgr, wih0t, whh0t, b0r, wih1t, whh1t, b1r, wfct,
      jnp.asarray(_U2))

    out = out + bfc[0]
    return out.reshape(2, N, TP, PK).transpose(0, 3, 1, 2).reshape(B, N, TP)
